# baseline (speedup 1.0000x reference)
import functools
import math

import jax
import jax.numpy as jnp
from jax import lax
from jax.experimental import pallas as pl
from jax.experimental.pallas import tpu as pltpu
from jax.experimental.pallas import tpu_sc as plsc

D = 1024
B = 4
S = 2048
NC = 2
NS = 16
NW = NC * NS
POS_PER_W = S // NW
CHUNK = 16
NCHUNK = POS_PER_W // CHUNK
LANES = 16
VPR = D // LANES
SCALE = math.sqrt(D)


def _pe_table(seq_len, d_model):
    pos = jnp.arange(seq_len, dtype=jnp.float32)[:, None]
    i = jnp.arange(0, d_model, 2, dtype=jnp.float32)[None, :]
    angle = pos / jnp.power(10000.0, i / d_model)
    pe = jnp.zeros((seq_len, d_model), dtype=jnp.float32)
    pe = pe.at[:, 0::2].set(jnp.sin(angle))
    pe = pe.at[:, 1::2].set(jnp.cos(angle))
    return pe


def _sc_body(tok_hbm, pe_hbm, table_hbm, out_hbm, idx_v, pe_v, rows_v, sem):
    wid = lax.axis_index("s") * NC + lax.axis_index("c")
    pos0 = wid * POS_PER_W

    def chunk_body(c, carry):
        pos = pos0 + c * CHUNK
        pltpu.sync_copy(pe_hbm.at[pl.ds(pos, CHUNK)], pe_v)
        for b in range(B):
            pltpu.sync_copy(tok_hbm.at[pl.ds(b * S + pos, CHUNK)],
                            idx_v.at[b])
        copies = [
            pltpu.async_copy(table_hbm.at[idx_v.at[b]], rows_v.at[b], sem)
            for b in range(B)
        ]
        for cp in copies:
            cp.wait()

        def row_body(p, carry2):
            def vec_body(v, carry3):
                col = v * LANES
                pe_reg = pe_v[p, pl.ds(col, LANES)]
                for b in range(B):
                    rows_v[b, p, pl.ds(col, LANES)] = (
                        rows_v[b, p, pl.ds(col, LANES)] * SCALE + pe_reg)
                return carry3
            return lax.fori_loop(0, VPR, vec_body, carry2)

        lax.fori_loop(0, CHUNK, row_body, carry)
        for b in range(B):
            pltpu.sync_copy(rows_v.at[b],
                            out_hbm.at[pl.ds(b * S + pos, CHUNK)])
        return carry

    lax.fori_loop(0, NCHUNK, chunk_body, 0)


def kernel(token_ids, table):
    tok_flat = token_ids.reshape(B * S).astype(jnp.int32)
    pe = _pe_table(S, D)
    mesh = plsc.VectorSubcoreMesh(core_axis_name="c", subcore_axis_name="s")
    run = pl.kernel(
        _sc_body,
        out_type=jax.ShapeDtypeStruct((B * S, D), jnp.float32),
        mesh=mesh,
        scratch_types=[
            pltpu.VMEM((B, CHUNK), jnp.int32),
            pltpu.VMEM((CHUNK, D), jnp.float32),
            pltpu.VMEM((B, CHUNK, D), jnp.float32),
            pltpu.SemaphoreType.DMA,
        ],
    )
    out = run(tok_flat, pe, table)
    return out.reshape(B, S, D)


# speedup vs baseline: 1.9432x; 1.9432x over previous
import math

import jax
import jax.numpy as jnp
from jax import lax
from jax.experimental import pallas as pl
from jax.experimental.pallas import tpu as pltpu
from jax.experimental.pallas import tpu_sc as plsc

D = 1024
B = 4
S = 2048
NC = 2
NS = 16
NW = NC * NS
POS_PER_W = S // NW
CHUNK = 8
NCHUNK = POS_PER_W // CHUNK
LANES = 16
VPR = D // LANES
SCALE = math.sqrt(D)


def _pe_table(seq_len, d_model):
    pos = jnp.arange(seq_len, dtype=jnp.float32)[:, None]
    i = jnp.arange(0, d_model, 2, dtype=jnp.float32)[None, :]
    angle = pos / jnp.power(10000.0, i / d_model)
    pe = jnp.zeros((seq_len, d_model), dtype=jnp.float32)
    pe = pe.at[:, 0::2].set(jnp.sin(angle))
    pe = pe.at[:, 1::2].set(jnp.cos(angle))
    return pe


def _sc_body(tok_hbm, pe_hbm, table_hbm, out_hbm,
             idx_v, pe_v, rows_v, in_sem0, in_sem1, out_sem):
    wid = lax.axis_index("s") * NC + lax.axis_index("c")
    pos0 = wid * POS_PER_W
    in_sems = (in_sem0, in_sem1)

    for b in range(B):
        pltpu.sync_copy(tok_hbm.at[pl.ds(b * S + pos0, POS_PER_W)],
                        idx_v.at[b])

    def start_chunk(c, slot):
        cps = [pltpu.async_copy(pe_hbm.at[pl.ds(pos0 + c * CHUNK, CHUNK)],
                                pe_v.at[slot], in_sems[slot])]
        for b in range(B):
            cps.append(pltpu.async_copy(
                table_hbm.at[idx_v.at[b, pl.ds(c * CHUNK, CHUNK)]],
                rows_v.at[slot, b], in_sems[slot]))
        return cps

    def start_out(c, slot):
        return [pltpu.async_copy(
            rows_v.at[slot, b],
            out_hbm.at[pl.ds(b * S + pos0 + c * CHUNK, CHUNK)], out_sem)
            for b in range(B)]

    def compute(slot):
        @plsc.parallel_loop(0, CHUNK * VPR, 1, unroll=2)
        def _(t):
            p = lax.shift_right_logical(t, 6)
            col = pl.multiple_of(
                lax.shift_left(lax.bitwise_and(t, VPR - 1), 4), LANES)
            pe_reg = pe_v[slot, p, pl.ds(col, LANES)]
            for b in range(B):
                rows_v[slot, b, p, pl.ds(col, LANES)] = (
                    rows_v[slot, b, p, pl.ds(col, LANES)] * SCALE + pe_reg)

    pending_in = {0: start_chunk(0, 0)}
    pending_out = {}
    for c in range(NCHUNK):
        slot = c & 1
        if c + 1 < NCHUNK:
            if c >= 1:
                for cp in pending_out.pop(c - 1):
                    cp.wait()
            pending_in[c + 1] = start_chunk(c + 1, slot ^ 1)
        for cp in pending_in.pop(c):
            cp.wait()
        compute(slot)
        pending_out[c] = start_out(c, slot)
    for c in sorted(pending_out):
        for cp in pending_out.pop(c):
            cp.wait()


def kernel(token_ids, table):
    tok_flat = token_ids.reshape(B * S).astype(jnp.int32)
    pe = _pe_table(S, D)
    mesh = plsc.VectorSubcoreMesh(core_axis_name="c", subcore_axis_name="s")
    run = pl.kernel(
        _sc_body,
        out_type=jax.ShapeDtypeStruct((B * S, D), jnp.float32),
        mesh=mesh,
        scratch_types=[
            pltpu.VMEM((B, POS_PER_W), jnp.int32),
            pltpu.VMEM((2, CHUNK, D), jnp.float32),
            pltpu.VMEM((2, B, CHUNK, D), jnp.float32),
            pltpu.SemaphoreType.DMA,
            pltpu.SemaphoreType.DMA,
            pltpu.SemaphoreType.DMA,
        ],
    )
    out = run(tok_flat, pe, table)
    return out.reshape(B, S, D)


# speedup vs baseline: 3.4033x; 1.7514x over previous
import math

import jax
import jax.numpy as jnp
import numpy as np
from jax import lax
from jax.experimental import pallas as pl
from jax.experimental.pallas import tpu as pltpu
from jax.experimental.pallas import tpu_sc as plsc

D = 1024
B = 4
S = 2048
NC = 2
NS = 16
NW = NC * NS
POS_PER_W = S // NW
CHUNK = 8
NCHUNK = POS_PER_W // CHUNK
LANES = 16
VPR = D // LANES
SCALE = math.sqrt(D)


def _pe_table(seq_len, d_model):
    pos = np.arange(seq_len, dtype=np.float32)[:, None]
    i = np.arange(0, d_model, 2, dtype=np.float32)[None, :]
    angle = (pos / np.power(np.float32(10000.0),
                            i / np.float32(d_model))).astype(np.float32)
    pe = np.zeros((seq_len, d_model), dtype=np.float32)
    pe[:, 0::2] = np.sin(angle)
    pe[:, 1::2] = np.cos(angle)
    return pe


_PE = _pe_table(S, D)


def _sc_body(tok_hbm, pe_hbm, table_hbm, out_hbm,
             idx_v, pe_v, rows_v, in_sem0, in_sem1, out_sem):
    wid = lax.axis_index("s") * NC + lax.axis_index("c")
    pos0 = wid * POS_PER_W
    in_sems = (in_sem0, in_sem1)

    for b in range(B):
        pltpu.sync_copy(tok_hbm.at[pl.ds(b * S + pos0, POS_PER_W)],
                        idx_v.at[b])

    def start_chunk(c, slot):
        cps = [pltpu.async_copy(pe_hbm.at[pl.ds(pos0 + c * CHUNK, CHUNK)],
                                pe_v.at[slot], in_sems[slot])]
        for b in range(B):
            cps.append(pltpu.async_copy(
                table_hbm.at[idx_v.at[b, pl.ds(c * CHUNK, CHUNK)]],
                rows_v.at[slot, b], in_sems[slot]))
        return cps

    def start_out(c, slot):
        return [pltpu.async_copy(
            rows_v.at[slot, b],
            out_hbm.at[pl.ds(b * S + pos0 + c * CHUNK, CHUNK)], out_sem)
            for b in range(B)]

    def compute(slot):
        @plsc.parallel_loop(0, CHUNK * VPR, 1, unroll=2)
        def _(t):
            p = lax.shift_right_logical(t, 6)
            col = pl.multiple_of(
                lax.shift_left(lax.bitwise_and(t, VPR - 1), 4), LANES)
            pe_reg = pe_v[slot, p, pl.ds(col, LANES)]
            for b in range(B):
                rows_v[slot, b, p, pl.ds(col, LANES)] = (
                    rows_v[slot, b, p, pl.ds(col, LANES)] * SCALE + pe_reg)

    pending_in = {0: start_chunk(0, 0)}
    pending_out = {}
    for c in range(NCHUNK):
        slot = c & 1
        if c + 1 < NCHUNK:
            if c >= 1:
                for cp in pending_out.pop(c - 1):
                    cp.wait()
            pending_in[c + 1] = start_chunk(c + 1, slot ^ 1)
        for cp in pending_in.pop(c):
            cp.wait()
        compute(slot)
        pending_out[c] = start_out(c, slot)
    for c in sorted(pending_out):
        for cp in pending_out.pop(c):
            cp.wait()


def kernel(token_ids, table):
    tok_flat = token_ids.reshape(B * S).astype(jnp.int32)
    pe = jnp.asarray(_PE)
    mesh = plsc.VectorSubcoreMesh(core_axis_name="c", subcore_axis_name="s")
    run = pl.kernel(
        _sc_body,
        out_type=jax.ShapeDtypeStruct((B * S, D), jnp.float32),
        mesh=mesh,
        scratch_types=[
            pltpu.VMEM((B, POS_PER_W), jnp.int32),
            pltpu.VMEM((2, CHUNK, D), jnp.float32),
            pltpu.VMEM((2, B, CHUNK, D), jnp.float32),
            pltpu.SemaphoreType.DMA,
            pltpu.SemaphoreType.DMA,
            pltpu.SemaphoreType.DMA,
        ],
    )
    out = run(tok_flat, pe, table)
    return out.reshape(B, S, D)
